# baseline (speedup 1.0000x reference)
import functools

import jax
import jax.numpy as jnp
from jax import lax
from jax.experimental import pallas as pl
from jax.experimental.pallas import tpu as pltpu
from jax.experimental.pallas import tpu_sc as plsc

N_NODES = 10000
HIDDEN = 128
ANNOT = 16
TIMESTEPS = 2
N_EDGES = 320000

NC = 2
NS = 16
HALF = HIDDEN // NC
CHUNK = 128
NCHUNK = 160
KGRP = 2
NGRP = NCHUNK // KGRP
NPAIR = NGRP // 2
EPT = CHUNK * NCHUNK
E_PAD = EPT * NS
ACC_ROWS = 10240
DEGW = 16
ROWS_PER_TILE = ACC_ROWS // NS


def _edge_agg_body(with_deg, *refs):
    if with_deg:
        (h_lo_hbm, h_hi_hbm, src_hbm, tgt_hbm, acc_out, deg_out,
         src_v, tgt_v, rows_v, ones_v, zdeg_v,
         acc_sh, deg_sh, gsem0, gsem1, ssem0, ssem1) = refs
    else:
        (h_lo_hbm, h_hi_hbm, src_hbm, tgt_hbm, acc_out,
         src_v, tgt_v, rows_v,
         acc_sh, gsem0, gsem1, ssem0, ssem1) = refs

    c = lax.axis_index("c")
    s = lax.axis_index("s")

    pltpu.sync_copy(src_hbm.at[s], src_v)
    pltpu.sync_copy(tgt_hbm.at[s], tgt_v)

    z16 = jnp.zeros((16,), jnp.float32)
    one16 = jnp.ones((16,), jnp.float32)

    def fill_row(i, carry):
        for v in range(HALF // 16):
            rows_v[0, 0, i, pl.ds(v * 16, 16)] = z16
        if with_deg:
            zdeg_v[i, pl.ds(0, 16)] = z16
            ones_v[i, pl.ds(0, 16)] = one16
        return carry

    lax.fori_loop(0, CHUNK, fill_row, 0)

    for b in range(ROWS_PER_TILE // CHUNK):
        base = s * ROWS_PER_TILE + b * CHUNK
        pltpu.sync_copy(rows_v.at[0, 0], acc_sh.at[pl.ds(base, CHUNK)])
        if with_deg:
            pltpu.sync_copy(zdeg_v, deg_sh.at[pl.ds(base, CHUNK)])
    plsc.subcore_barrier()

    h_hbm = (h_lo_hbm, h_hi_hbm)
    gsem = (gsem0, gsem1)
    ssem = (ssem0, ssem1)

    def fire_gathers(g, a):
        for b in range(KGRP):
            for ci in range(NC):
                @pl.when(c == ci)
                def _():
                    pltpu.async_copy(h_hbm[ci].at[src_v.at[g * KGRP + b]],
                                     rows_v.at[a, b], gsem[a])

    def wait_gathers(g, a):
        for b in range(KGRP):
            for ci in range(NC):
                @pl.when(c == ci)
                def _():
                    pltpu.make_async_copy(h_hbm[ci].at[src_v.at[g * KGRP + b]],
                                          rows_v.at[a, b], gsem[a]).wait()

    def fire_scatters(g, a):
        for b in range(KGRP):
            j = g * KGRP + b
            pltpu.async_copy(rows_v.at[a, b], acc_sh.at[tgt_v.at[j]],
                             ssem[a], add=True)

            if with_deg:
                @pl.when(c == (b % NC))
                def _():
                    pltpu.async_copy(ones_v, deg_sh.at[tgt_v.at[j]],
                                     ssem[a], add=True)

    def wait_scatters(g, a):
        for b in range(KGRP):
            j = g * KGRP + b
            pltpu.make_async_copy(rows_v.at[a, b], acc_sh.at[tgt_v.at[j]],
                                  ssem[a]).wait()

            if with_deg:
                @pl.when(c == (b % NC))
                def _():
                    pltpu.make_async_copy(ones_v, deg_sh.at[tgt_v.at[j]],
                                          ssem[a]).wait()

    fire_gathers(0, 0)

    def pair_body(p, carry):
        g0 = 2 * p
        g1 = 2 * p + 1
        wait_gathers(g0, 0)
        fire_scatters(g0, 0)

        @pl.when(p > 0)
        def _():
            wait_scatters(g1 - 2, 1)
        fire_gathers(g1, 1)
        wait_gathers(g1, 1)
        fire_scatters(g1, 1)
        wait_scatters(g0, 0)

        @pl.when(p < NPAIR - 1)
        def _():
            fire_gathers(g1 + 1, 0)
        return carry

    lax.fori_loop(0, NPAIR, pair_body, 0)
    wait_scatters(NGRP - 1, 1)
    plsc.subcore_barrier()

    sl = pl.ds(s * ROWS_PER_TILE, ROWS_PER_TILE)
    pltpu.sync_copy(acc_sh.at[sl], acc_out.at[c, sl])

    if with_deg:
        pltpu.sync_copy(deg_sh.at[sl], deg_out.at[c, sl])


def _make_edge_agg(with_deg):
    out_type = [jax.ShapeDtypeStruct((NC, ACC_ROWS, HALF), jnp.float32)]
    scratch = [
        pltpu.VMEM((NCHUNK, CHUNK), jnp.int32),
        pltpu.VMEM((NCHUNK, CHUNK), jnp.int32),
        pltpu.VMEM((2, KGRP, CHUNK, HALF), jnp.float32),
    ]
    if with_deg:
        out_type.append(jax.ShapeDtypeStruct((NC, ACC_ROWS, DEGW), jnp.float32))
        scratch += [
            pltpu.VMEM((CHUNK, DEGW), jnp.float32),
            pltpu.VMEM((CHUNK, DEGW), jnp.float32),
        ]
    scratch.append(pltpu.VMEM_SHARED((ACC_ROWS, HALF), jnp.float32))
    if with_deg:
        scratch.append(pltpu.VMEM_SHARED((ACC_ROWS, DEGW), jnp.float32))
    scratch += [
        pltpu.SemaphoreType.DMA,
        pltpu.SemaphoreType.DMA,
        pltpu.SemaphoreType.DMA,
        pltpu.SemaphoreType.DMA,
    ]
    return pl.kernel(
        functools.partial(_edge_agg_body, with_deg),
        out_type=tuple(out_type) if with_deg else out_type[0],
        mesh=plsc.VectorSubcoreMesh(core_axis_name="c", subcore_axis_name="s"),
        compiler_params=pltpu.CompilerParams(use_tc_tiling_on_sc=False),
        scratch_types=scratch,
    )


_edge_agg_deg = _make_edge_agg(True)
_edge_agg_nodeg = _make_edge_agg(False)


BLK = 1000


def _split_out(hnew, o_ref, olo_ref, ohi_ref):
    o_ref[...] = hnew
    olo_ref[...] = hnew[:, :HALF]
    ohi_ref[...] = hnew[:, HALF:]


def _h0_body(x_ref, a_ref, wx_ref, wa_ref, b_ref, o_ref, olo_ref, ohi_ref):
    h0 = (jnp.dot(x_ref[...], wx_ref[...], preferred_element_type=jnp.float32)
          + jnp.dot(a_ref[...], wa_ref[...], preferred_element_type=jnp.float32)
          + b_ref[...])
    _split_out(h0, o_ref, olo_ref, ohi_ref)


_SPLIT_OUT_SHAPE = (
    jax.ShapeDtypeStruct((N_NODES, HIDDEN), jnp.float32),
    jax.ShapeDtypeStruct((N_NODES, HALF), jnp.float32),
    jax.ShapeDtypeStruct((N_NODES, HALF), jnp.float32),
)
_SPLIT_OUT_SPECS = [
    pl.BlockSpec((BLK, HIDDEN), lambda i: (i, 0)),
    pl.BlockSpec((BLK, HALF), lambda i: (i, 0)),
    pl.BlockSpec((BLK, HALF), lambda i: (i, 0)),
]


def _h0_call(x, annot, wx, wa, b):
    return pl.pallas_call(
        _h0_body,
        grid=(N_NODES // BLK,),
        in_specs=[
            pl.BlockSpec((BLK, HIDDEN), lambda i: (i, 0)),
            pl.BlockSpec((BLK, ANNOT), lambda i: (i, 0)),
            pl.BlockSpec((HIDDEN, HIDDEN), lambda i: (0, 0)),
            pl.BlockSpec((ANNOT, HIDDEN), lambda i: (0, 0)),
            pl.BlockSpec((1, HIDDEN), lambda i: (0, 0)),
        ],
        out_specs=_SPLIT_OUT_SPECS,
        out_shape=_SPLIT_OUT_SHAPE,
    )(x, annot, wx, wa, b)


def _gh_body(h_ref, whh_ref, bhh_ref, o_ref):
    o_ref[...] = (jnp.dot(h_ref[...], whh_ref[...],
                          preferred_element_type=jnp.float32) + bhh_ref[...])


def _gh_call(h, whh, bhh):
    return pl.pallas_call(
        _gh_body,
        grid=(N_NODES // BLK,),
        in_specs=[
            pl.BlockSpec((BLK, HIDDEN), lambda i: (i, 0)),
            pl.BlockSpec((HIDDEN, 3 * HIDDEN), lambda i: (0, 0)),
            pl.BlockSpec((1, 3 * HIDDEN), lambda i: (0, 0)),
        ],
        out_specs=pl.BlockSpec((BLK, 3 * HIDDEN), lambda i: (i, 0)),
        out_shape=jax.ShapeDtypeStruct((N_NODES, 3 * HIDDEN), jnp.float32),
    )(h, whh, bhh)


def _gru_body(acc_ref, dega_ref, h_ref, gh_ref, wmlo_ref, wmhi_ref, bmsg_ref,
              wih_ref, bih_ref, o_ref, olo_ref, ohi_ref):
    deg = dega_ref[0, :, :1] + dega_ref[1, :, :1]
    inc = (jnp.dot(acc_ref[0], wmlo_ref[...], preferred_element_type=jnp.float32)
           + jnp.dot(acc_ref[1], wmhi_ref[...], preferred_element_type=jnp.float32)
           + deg * bmsg_ref[...])
    h = h_ref[...]
    gi = jnp.dot(inc, wih_ref[...], preferred_element_type=jnp.float32) + bih_ref[...]
    gh = gh_ref[...]
    r = jax.nn.sigmoid(gi[:, :HIDDEN] + gh[:, :HIDDEN])
    z = jax.nn.sigmoid(gi[:, HIDDEN:2 * HIDDEN] + gh[:, HIDDEN:2 * HIDDEN])
    n = jnp.tanh(gi[:, 2 * HIDDEN:] + r * gh[:, 2 * HIDDEN:])
    _split_out((1.0 - z) * n + z * h, o_ref, olo_ref, ohi_ref)


def _gru_call(acc, dega, h, gh, wmlo, wmhi, bmsg, wih, bih):
    return pl.pallas_call(
        _gru_body,
        grid=(N_NODES // BLK,),
        in_specs=[
            pl.BlockSpec((NC, BLK, HALF), lambda i: (0, i, 0)),
            pl.BlockSpec((NC, BLK, DEGW), lambda i: (0, i, 0)),
            pl.BlockSpec((BLK, HIDDEN), lambda i: (i, 0)),
            pl.BlockSpec((BLK, 3 * HIDDEN), lambda i: (i, 0)),
            pl.BlockSpec((HALF, HIDDEN), lambda i: (0, 0)),
            pl.BlockSpec((HALF, HIDDEN), lambda i: (0, 0)),
            pl.BlockSpec((1, HIDDEN), lambda i: (0, 0)),
            pl.BlockSpec((HIDDEN, 3 * HIDDEN), lambda i: (0, 0)),
            pl.BlockSpec((1, 3 * HIDDEN), lambda i: (0, 0)),
        ],
        out_specs=_SPLIT_OUT_SPECS,
        out_shape=_SPLIT_OUT_SHAPE,
    )(acc, dega, h, gh, wmlo, wmhi, bmsg, wih, bih)


def kernel(initial_node_representation, annotations, adjacency_lists,
           W_hidden, b_hidden, W_msg, b_msg, W_ih, b_ih, W_hh, b_hh):
    src = adjacency_lists[:, 0].astype(jnp.int32)
    tgt = adjacency_lists[:, 1].astype(jnp.int32)

    npad = E_PAD - N_EDGES
    ar = jnp.arange(npad, dtype=jnp.int32)
    pad_src = (ar * 37) % N_NODES
    pad_tgt = N_NODES + ar % (ACC_ROWS - N_NODES)
    src3 = jnp.concatenate([src, pad_src]).reshape(NS, NCHUNK, CHUNK)
    tgt3 = jnp.concatenate([tgt, pad_tgt]).reshape(NS, NCHUNK, CHUNK)

    wx = W_hidden[:HIDDEN]
    wa = W_hidden[HIDDEN:]
    wmlo = W_msg[:HALF]
    wmhi = W_msg[HALF:]
    bh2 = b_hidden.reshape(1, HIDDEN)
    bm2 = b_msg.reshape(1, HIDDEN)
    bih2 = b_ih.reshape(1, 3 * HIDDEN)
    bhh2 = b_hh.reshape(1, 3 * HIDDEN)

    h, h_lo, h_hi = _h0_call(initial_node_representation, annotations, wx, wa, bh2)
    dega = None
    for t in range(TIMESTEPS):
        gh = _gh_call(h, W_hh, bhh2)
        if t == 0:
            acc, dega = _edge_agg_deg(h_lo, h_hi, src3, tgt3)
        else:
            acc = _edge_agg_nodeg(h_lo, h_hi, src3, tgt3)
        h, h_lo, h_hi = _gru_call(acc, dega, h, gh, wmlo, wmhi, bm2,
                                  W_ih, bih2)
    return h


# speedup vs baseline: 1.0872x; 1.0872x over previous
import functools

import jax
import jax.numpy as jnp
from jax import lax
from jax.experimental import pallas as pl
from jax.experimental.pallas import tpu as pltpu
from jax.experimental.pallas import tpu_sc as plsc

N_NODES = 10000
HIDDEN = 128
ANNOT = 16
TIMESTEPS = 2
N_EDGES = 320000

NC = 2
NS = 16
HALF = HIDDEN // NC
CHUNK = 128
NCHUNK = 160
KGRP = 2
NGRP = NCHUNK // KGRP
NPAIR = NGRP // 2
EPT = CHUNK * NCHUNK
E_PAD = EPT * NS
ACC_ROWS = 10240
DEGW = 16
ROWS_PER_TILE = ACC_ROWS // NS


def _edge_agg_body(with_deg, *refs):
    if with_deg:
        (h2_hbm, srcl_hbm, srch_hbm, tgt_hbm, acc_out, deg_out,
         src_v, tgt_v, rows_v, ones_v, zdeg_v,
         acc_sh, deg_sh, gsem0, gsem1, ssem0, ssem1) = refs
    else:
        (h2_hbm, srcl_hbm, srch_hbm, tgt_hbm, acc_out,
         src_v, tgt_v, rows_v,
         acc_sh, gsem0, gsem1, ssem0, ssem1) = refs

    c = lax.axis_index("c")
    s = lax.axis_index("s")

    for ci, ref in enumerate((srcl_hbm, srch_hbm)):
        @pl.when(c == ci)
        def _(ref=ref):
            pltpu.sync_copy(ref.at[s], src_v)
    pltpu.sync_copy(tgt_hbm.at[s], tgt_v)

    z16 = jnp.zeros((16,), jnp.float32)
    one16 = jnp.ones((16,), jnp.float32)

    def fill_row(i, carry):
        for v in range(HALF // 16):
            rows_v[0, 0, i, pl.ds(v * 16, 16)] = z16
        if with_deg:
            zdeg_v[i, pl.ds(0, 16)] = z16
            ones_v[i, pl.ds(0, 16)] = one16
        return carry

    lax.fori_loop(0, CHUNK, fill_row, 0)

    for b in range(ROWS_PER_TILE // CHUNK):
        base = s * ROWS_PER_TILE + b * CHUNK
        pltpu.sync_copy(rows_v.at[0, 0], acc_sh.at[pl.ds(base, CHUNK)])
        if with_deg:
            pltpu.sync_copy(zdeg_v, deg_sh.at[pl.ds(base, CHUNK)])
    plsc.subcore_barrier()

    gsem = (gsem0, gsem1)
    ssem = (ssem0, ssem1)

    def fire_gathers(g, a):
        for b in range(KGRP):
            pltpu.async_copy(h2_hbm.at[src_v.at[g * KGRP + b]],
                             rows_v.at[a, b], gsem[a])

    def wait_gathers(g, a):
        for b in range(KGRP):
            pltpu.make_async_copy(h2_hbm.at[src_v.at[g * KGRP + b]],
                                  rows_v.at[a, b], gsem[a]).wait()

    def fire_scatters(g, a):
        for b in range(KGRP):
            j = g * KGRP + b
            pltpu.async_copy(rows_v.at[a, b], acc_sh.at[tgt_v.at[j]],
                             ssem[a], add=True)

            if with_deg:
                @pl.when(c == (b % NC))
                def _():
                    pltpu.async_copy(ones_v, deg_sh.at[tgt_v.at[j]],
                                     ssem[a], add=True)

    def wait_scatters(g, a):
        for b in range(KGRP):
            j = g * KGRP + b
            pltpu.make_async_copy(rows_v.at[a, b], acc_sh.at[tgt_v.at[j]],
                                  ssem[a]).wait()

            if with_deg:
                @pl.when(c == (b % NC))
                def _():
                    pltpu.make_async_copy(ones_v, deg_sh.at[tgt_v.at[j]],
                                          ssem[a]).wait()

    fire_gathers(0, 0)

    def pair_body(p, carry):
        g0 = 2 * p
        g1 = 2 * p + 1
        wait_gathers(g0, 0)
        fire_scatters(g0, 0)

        @pl.when(p > 0)
        def _():
            wait_scatters(g1 - 2, 1)
        fire_gathers(g1, 1)
        wait_gathers(g1, 1)
        fire_scatters(g1, 1)
        wait_scatters(g0, 0)

        @pl.when(p < NPAIR - 1)
        def _():
            fire_gathers(g1 + 1, 0)
        return carry

    lax.fori_loop(0, NPAIR, pair_body, 0)
    wait_scatters(NGRP - 1, 1)
    plsc.subcore_barrier()

    sl = pl.ds(s * ROWS_PER_TILE, ROWS_PER_TILE)
    pltpu.sync_copy(acc_sh.at[sl], acc_out.at[c, sl])

    if with_deg:
        pltpu.sync_copy(deg_sh.at[sl], deg_out.at[c, sl])


def _make_edge_agg(with_deg):
    out_type = [jax.ShapeDtypeStruct((NC, ACC_ROWS, HALF), jnp.float32)]
    scratch = [
        pltpu.VMEM((NCHUNK, CHUNK), jnp.int32),
        pltpu.VMEM((NCHUNK, CHUNK), jnp.int32),
        pltpu.VMEM((2, KGRP, CHUNK, HALF), jnp.float32),
    ]
    if with_deg:
        out_type.append(jax.ShapeDtypeStruct((NC, ACC_ROWS, DEGW), jnp.float32))
        scratch += [
            pltpu.VMEM((CHUNK, DEGW), jnp.float32),
            pltpu.VMEM((CHUNK, DEGW), jnp.float32),
        ]
    scratch.append(pltpu.VMEM_SHARED((ACC_ROWS, HALF), jnp.float32))
    if with_deg:
        scratch.append(pltpu.VMEM_SHARED((ACC_ROWS, DEGW), jnp.float32))
    scratch += [
        pltpu.SemaphoreType.DMA,
        pltpu.SemaphoreType.DMA,
        pltpu.SemaphoreType.DMA,
        pltpu.SemaphoreType.DMA,
    ]
    return pl.kernel(
        functools.partial(_edge_agg_body, with_deg),
        out_type=tuple(out_type) if with_deg else out_type[0],
        mesh=plsc.VectorSubcoreMesh(core_axis_name="c", subcore_axis_name="s"),
        compiler_params=pltpu.CompilerParams(use_tc_tiling_on_sc=False),
        scratch_types=scratch,
    )


_edge_agg_deg = _make_edge_agg(True)
_edge_agg_nodeg = _make_edge_agg(False)


BLK = 1000


def _h0_body(x_ref, a_ref, wx_ref, wa_ref, b_ref, o_ref):
    o_ref[...] = (jnp.dot(x_ref[...], wx_ref[...], preferred_element_type=jnp.float32)
                  + jnp.dot(a_ref[...], wa_ref[...], preferred_element_type=jnp.float32)
                  + b_ref[...])


def _h0_call(x, annot, wx, wa, b):
    return pl.pallas_call(
        _h0_body,
        grid=(N_NODES // BLK,),
        in_specs=[
            pl.BlockSpec((BLK, HIDDEN), lambda i: (i, 0)),
            pl.BlockSpec((BLK, ANNOT), lambda i: (i, 0)),
            pl.BlockSpec((HIDDEN, HIDDEN), lambda i: (0, 0)),
            pl.BlockSpec((ANNOT, HIDDEN), lambda i: (0, 0)),
            pl.BlockSpec((1, HIDDEN), lambda i: (0, 0)),
        ],
        out_specs=pl.BlockSpec((BLK, HIDDEN), lambda i: (i, 0)),
        out_shape=jax.ShapeDtypeStruct((N_NODES, HIDDEN), jnp.float32),
    )(x, annot, wx, wa, b)


def _gru_body(acc_ref, dega_ref, h_ref, wmlo_ref, wmhi_ref, bmsg_ref,
              wih_ref, bih_ref, whh_ref, bhh_ref, o_ref):
    deg = dega_ref[0, :, :1] + dega_ref[1, :, :1]
    inc = (jnp.dot(acc_ref[0], wmlo_ref[...], preferred_element_type=jnp.float32)
           + jnp.dot(acc_ref[1], wmhi_ref[...], preferred_element_type=jnp.float32)
           + deg * bmsg_ref[...])
    h = h_ref[...]
    gi = jnp.dot(inc, wih_ref[...], preferred_element_type=jnp.float32) + bih_ref[...]
    gh = jnp.dot(h, whh_ref[...], preferred_element_type=jnp.float32) + bhh_ref[...]
    r = jax.nn.sigmoid(gi[:, :HIDDEN] + gh[:, :HIDDEN])
    z = jax.nn.sigmoid(gi[:, HIDDEN:2 * HIDDEN] + gh[:, HIDDEN:2 * HIDDEN])
    n = jnp.tanh(gi[:, 2 * HIDDEN:] + r * gh[:, 2 * HIDDEN:])
    o_ref[...] = (1.0 - z) * n + z * h


def _gru_call(acc, dega, h, wmlo, wmhi, bmsg, wih, bih, whh, bhh):
    return pl.pallas_call(
        _gru_body,
        grid=(N_NODES // BLK,),
        in_specs=[
            pl.BlockSpec((NC, BLK, HALF), lambda i: (0, i, 0)),
            pl.BlockSpec((NC, BLK, DEGW), lambda i: (0, i, 0)),
            pl.BlockSpec((BLK, HIDDEN), lambda i: (i, 0)),
            pl.BlockSpec((HALF, HIDDEN), lambda i: (0, 0)),
            pl.BlockSpec((HALF, HIDDEN), lambda i: (0, 0)),
            pl.BlockSpec((1, HIDDEN), lambda i: (0, 0)),
            pl.BlockSpec((HIDDEN, 3 * HIDDEN), lambda i: (0, 0)),
            pl.BlockSpec((1, 3 * HIDDEN), lambda i: (0, 0)),
            pl.BlockSpec((HIDDEN, 3 * HIDDEN), lambda i: (0, 0)),
            pl.BlockSpec((1, 3 * HIDDEN), lambda i: (0, 0)),
        ],
        out_specs=pl.BlockSpec((BLK, HIDDEN), lambda i: (i, 0)),
        out_shape=jax.ShapeDtypeStruct((N_NODES, HIDDEN), jnp.float32),
    )(acc, dega, h, wmlo, wmhi, bmsg, wih, bih, whh, bhh)


def kernel(initial_node_representation, annotations, adjacency_lists,
           W_hidden, b_hidden, W_msg, b_msg, W_ih, b_ih, W_hh, b_hh):
    src = adjacency_lists[:, 0].astype(jnp.int32)
    tgt = adjacency_lists[:, 1].astype(jnp.int32)

    npad = E_PAD - N_EDGES
    ar = jnp.arange(npad, dtype=jnp.int32)
    pad_src = (ar * 37) % N_NODES
    pad_tgt = N_NODES + ar % (ACC_ROWS - N_NODES)
    src2 = 2 * jnp.concatenate([src, pad_src])
    srcl3 = src2.reshape(NS, NCHUNK, CHUNK)
    srch3 = (src2 + 1).reshape(NS, NCHUNK, CHUNK)
    tgt3 = jnp.concatenate([tgt, pad_tgt]).reshape(NS, NCHUNK, CHUNK)

    wx = W_hidden[:HIDDEN]
    wa = W_hidden[HIDDEN:]
    wmlo = W_msg[:HALF]
    wmhi = W_msg[HALF:]
    bh2 = b_hidden.reshape(1, HIDDEN)
    bm2 = b_msg.reshape(1, HIDDEN)
    bih2 = b_ih.reshape(1, 3 * HIDDEN)
    bhh2 = b_hh.reshape(1, 3 * HIDDEN)

    h = _h0_call(initial_node_representation, annotations, wx, wa, bh2)
    dega = None
    for t in range(TIMESTEPS):
        h2 = h.reshape(2 * N_NODES, HALF)
        if t == 0:
            acc, dega = _edge_agg_deg(h2, srcl3, srch3, tgt3)
        else:
            acc = _edge_agg_nodeg(h2, srcl3, srch3, tgt3)
        h = _gru_call(acc, dega, h, wmlo, wmhi, bm2, W_ih, bih2, W_hh, bhh2)
    return h


# speedup vs baseline: 1.1529x; 1.0604x over previous
import functools

import jax
import jax.numpy as jnp
from jax import lax
from jax.experimental import pallas as pl
from jax.experimental.pallas import tpu as pltpu
from jax.experimental.pallas import tpu_sc as plsc

N_NODES = 10000
HIDDEN = 128
ANNOT = 16
TIMESTEPS = 2
N_EDGES = 320000

NC = 2
NS = 16
HALF = HIDDEN // NC
CHUNK = 128
NCHUNK = 160
KGRP = 2
NGRP = NCHUNK // KGRP
NPAIR = NGRP // 2
EPT = CHUNK * NCHUNK
E_PAD = EPT * NS
ACC_ROWS = 10240
DEGW = 16
ROWS_PER_TILE = ACC_ROWS // NS


def _edge_agg_body(with_deg, *refs):
    if with_deg:
        (h2_hbm, srcl_hbm, srch_hbm, tgt_hbm, acc_out, deg_out,
         src_v, tgt_v, rows_v, ones_v, zdeg_v,
         acc_sh, deg_sh, gsem0, gsem1, ssem0, ssem1) = refs
    else:
        (h2_hbm, srcl_hbm, srch_hbm, tgt_hbm, acc_out,
         src_v, tgt_v, rows_v,
         acc_sh, gsem0, gsem1, ssem0, ssem1) = refs

    c = lax.axis_index("c")
    s = lax.axis_index("s")

    for ci, ref in enumerate((srcl_hbm, srch_hbm)):
        @pl.when(c == ci)
        def _(ref=ref):
            pltpu.sync_copy(ref.at[s], src_v)
    pltpu.sync_copy(tgt_hbm.at[s], tgt_v)

    z16 = jnp.zeros((16,), jnp.float32)
    one16 = jnp.ones((16,), jnp.float32)

    def fill_row(i, carry):
        for v in range(HALF // 16):
            rows_v[0, 0, i, pl.ds(v * 16, 16)] = z16
        if with_deg:
            zdeg_v[i, pl.ds(0, 16)] = z16
            ones_v[i, pl.ds(0, 16)] = one16
        return carry

    lax.fori_loop(0, CHUNK, fill_row, 0)

    for b in range(ROWS_PER_TILE // CHUNK):
        base = s * ROWS_PER_TILE + b * CHUNK
        pltpu.sync_copy(rows_v.at[0, 0], acc_sh.at[pl.ds(base, CHUNK)])
        if with_deg:
            pltpu.sync_copy(zdeg_v, deg_sh.at[pl.ds(base, CHUNK)])
    plsc.subcore_barrier()

    gsem = (gsem0, gsem1)
    ssem = (ssem0, ssem1)

    def fire_gathers(g, a):
        for b in range(KGRP):
            pltpu.async_copy(h2_hbm.at[src_v.at[g * KGRP + b]],
                             rows_v.at[a, b], gsem[a])

    def wait_gathers(g, a):
        for b in range(KGRP):
            pltpu.make_async_copy(h2_hbm.at[src_v.at[g * KGRP + b]],
                                  rows_v.at[a, b], gsem[a]).wait()

    def fire_scatters(g, a):
        for b in range(KGRP):
            j = g * KGRP + b
            pltpu.async_copy(rows_v.at[a, b], acc_sh.at[tgt_v.at[j]],
                             ssem[a], add=True)

            if with_deg:
                @pl.when(c == (b % NC))
                def _():
                    pltpu.async_copy(ones_v, deg_sh.at[tgt_v.at[j]],
                                     ssem[a], add=True)

    def wait_scatters(g, a):
        for b in range(KGRP):
            j = g * KGRP + b
            pltpu.make_async_copy(rows_v.at[a, b], acc_sh.at[tgt_v.at[j]],
                                  ssem[a]).wait()

            if with_deg:
                @pl.when(c == (b % NC))
                def _():
                    pltpu.make_async_copy(ones_v, deg_sh.at[tgt_v.at[j]],
                                          ssem[a]).wait()

    fire_gathers(0, 0)

    def pair_body(p, carry):
        g0 = 2 * p
        g1 = 2 * p + 1
        wait_gathers(g0, 0)
        fire_scatters(g0, 0)

        @pl.when(p > 0)
        def _():
            wait_scatters(g1 - 2, 1)
        fire_gathers(g1, 1)
        wait_gathers(g1, 1)
        fire_scatters(g1, 1)
        wait_scatters(g0, 0)

        @pl.when(p < NPAIR - 1)
        def _():
            fire_gathers(g1 + 1, 0)
        return carry

    lax.fori_loop(0, NPAIR, pair_body, 0)
    wait_scatters(NGRP - 1, 1)
    plsc.subcore_barrier()

    sl = pl.ds(s * ROWS_PER_TILE, ROWS_PER_TILE)
    pltpu.sync_copy(acc_sh.at[sl], acc_out.at[c, sl])

    if with_deg:
        pltpu.sync_copy(deg_sh.at[sl], deg_out.at[c, sl])


def _make_edge_agg(with_deg):
    out_type = [jax.ShapeDtypeStruct((NC, ACC_ROWS, HALF), jnp.float32)]
    scratch = [
        pltpu.VMEM((NCHUNK, CHUNK), jnp.int32),
        pltpu.VMEM((NCHUNK, CHUNK), jnp.int32),
        pltpu.VMEM((2, KGRP, CHUNK, HALF), jnp.float32),
    ]
    if with_deg:
        out_type.append(jax.ShapeDtypeStruct((NC, ACC_ROWS, DEGW), jnp.float32))
        scratch += [
            pltpu.VMEM((CHUNK, DEGW), jnp.float32),
            pltpu.VMEM((CHUNK, DEGW), jnp.float32),
        ]
    scratch.append(pltpu.VMEM_SHARED((ACC_ROWS, HALF), jnp.float32))
    if with_deg:
        scratch.append(pltpu.VMEM_SHARED((ACC_ROWS, DEGW), jnp.float32))
    scratch += [
        pltpu.SemaphoreType.DMA,
        pltpu.SemaphoreType.DMA,
        pltpu.SemaphoreType.DMA,
        pltpu.SemaphoreType.DMA,
    ]
    return pl.kernel(
        functools.partial(_edge_agg_body, with_deg),
        out_type=tuple(out_type) if with_deg else out_type[0],
        mesh=plsc.VectorSubcoreMesh(core_axis_name="c", subcore_axis_name="s"),
        compiler_params=pltpu.CompilerParams(use_tc_tiling_on_sc=False),
        scratch_types=scratch,
    )


_edge_agg_deg = _make_edge_agg(True)
_edge_agg_nodeg = _make_edge_agg(False)


BLK = 2000


def _h0_body(x_ref, a_ref, wx_ref, wa_ref, b_ref, o_ref):
    o_ref[...] = (jnp.dot(x_ref[...], wx_ref[...], preferred_element_type=jnp.float32)
                  + jnp.dot(a_ref[...], wa_ref[...], preferred_element_type=jnp.float32)
                  + b_ref[...])


def _h0_call(x, annot, wx, wa, b):
    return pl.pallas_call(
        _h0_body,
        grid=(N_NODES // BLK,),
        in_specs=[
            pl.BlockSpec((BLK, HIDDEN), lambda i: (i, 0)),
            pl.BlockSpec((BLK, ANNOT), lambda i: (i, 0)),
            pl.BlockSpec((HIDDEN, HIDDEN), lambda i: (0, 0)),
            pl.BlockSpec((ANNOT, HIDDEN), lambda i: (0, 0)),
            pl.BlockSpec((1, HIDDEN), lambda i: (0, 0)),
        ],
        out_specs=pl.BlockSpec((BLK, HIDDEN), lambda i: (i, 0)),
        out_shape=jax.ShapeDtypeStruct((N_NODES, HIDDEN), jnp.float32),
    )(x, annot, wx, wa, b)


def _gru_body(acc_ref, dega_ref, h_ref, wme_ref, wmo_ref, bmsg_ref,
              wih_ref, bih_ref, whh_ref, bhh_ref, o_ref):
    deg = dega_ref[0, :, :1] + dega_ref[1, :, :1]
    q_even =(jnp.dot(acc_ref[0], wme_ref[0], preferred_element_type=jnp.float32)
              + jnp.dot(acc_ref[1], wme_ref[1], preferred_element_type=jnp.float32))
    q_odd = (jnp.dot(acc_ref[0], wmo_ref[0], preferred_element_type=jnp.float32)
             + jnp.dot(acc_ref[1], wmo_ref[1], preferred_element_type=jnp.float32))
    inc = (jnp.stack([q_even, q_odd], axis=1).reshape(BLK, HIDDEN)
           + deg * bmsg_ref[...])
    h = h_ref[...]
    gi = jnp.dot(inc, wih_ref[...], preferred_element_type=jnp.float32) + bih_ref[...]
    gh = jnp.dot(h, whh_ref[...], preferred_element_type=jnp.float32) + bhh_ref[...]
    r = jax.nn.sigmoid(gi[:, :HIDDEN] + gh[:, :HIDDEN])
    z = jax.nn.sigmoid(gi[:, HIDDEN:2 * HIDDEN] + gh[:, HIDDEN:2 * HIDDEN])
    n = jnp.tanh(gi[:, 2 * HIDDEN:] + r * gh[:, 2 * HIDDEN:])
    o_ref[...] = (1.0 - z) * n + z * h


def _gru_call(acc, dega, h, wme, wmo, bmsg, wih, bih, whh, bhh):
    return pl.pallas_call(
        _gru_body,
        grid=(N_NODES // BLK,),
        in_specs=[
            pl.BlockSpec((NC, BLK // 2, HIDDEN), lambda i: (0, i, 0)),
            pl.BlockSpec((NC, BLK, DEGW), lambda i: (0, i, 0)),
            pl.BlockSpec((BLK, HIDDEN), lambda i: (i, 0)),
            pl.BlockSpec((NC, HIDDEN, HIDDEN), lambda i: (0, 0, 0)),
            pl.BlockSpec((NC, HIDDEN, HIDDEN), lambda i: (0, 0, 0)),
            pl.BlockSpec((1, HIDDEN), lambda i: (0, 0)),
            pl.BlockSpec((HIDDEN, 3 * HIDDEN), lambda i: (0, 0)),
            pl.BlockSpec((1, 3 * HIDDEN), lambda i: (0, 0)),
            pl.BlockSpec((HIDDEN, 3 * HIDDEN), lambda i: (0, 0)),
            pl.BlockSpec((1, 3 * HIDDEN), lambda i: (0, 0)),
        ],
        out_specs=pl.BlockSpec((BLK, HIDDEN), lambda i: (i, 0)),
        out_shape=jax.ShapeDtypeStruct((N_NODES, HIDDEN), jnp.float32),
    )(acc, dega, h, wme, wmo, bmsg, wih, bih, whh, bhh)


def kernel(initial_node_representation, annotations, adjacency_lists,
           W_hidden, b_hidden, W_msg, b_msg, W_ih, b_ih, W_hh, b_hh):
    src = adjacency_lists[:, 0].astype(jnp.int32)
    tgt = adjacency_lists[:, 1].astype(jnp.int32)

    npad = E_PAD - N_EDGES
    ar = jnp.arange(npad, dtype=jnp.int32)
    pad_src = (ar * 37) % N_NODES
    pad_tgt = N_NODES + ar % (ACC_ROWS - N_NODES)
    src2 = 2 * jnp.concatenate([src, pad_src])
    srcl3 = src2.reshape(NS, NCHUNK, CHUNK)
    srch3 = (src2 + 1).reshape(NS, NCHUNK, CHUNK)
    tgt3 = jnp.concatenate([tgt, pad_tgt]).reshape(NS, NCHUNK, CHUNK)

    wx = W_hidden[:HIDDEN]
    wa = W_hidden[HIDDEN:]
    z64 = jnp.zeros((HALF, HIDDEN), jnp.float32)
    wme = jnp.stack([jnp.concatenate([W_msg[:HALF], z64]),
                     jnp.concatenate([W_msg[HALF:], z64])])
    wmo = jnp.stack([jnp.concatenate([z64, W_msg[:HALF]]),
                     jnp.concatenate([z64, W_msg[HALF:]])])
    bh2 = b_hidden.reshape(1, HIDDEN)
    bm2 = b_msg.reshape(1, HIDDEN)
    bih2 = b_ih.reshape(1, 3 * HIDDEN)
    bhh2 = b_hh.reshape(1, 3 * HIDDEN)

    h = _h0_call(initial_node_representation, annotations, wx, wa, bh2)
    dega = None
    for t in range(TIMESTEPS):
        h2 = h.reshape(2 * N_NODES, HALF)
        if t == 0:
            acc, dega = _edge_agg_deg(h2, srcl3, srch3, tgt3)
        else:
            acc = _edge_agg_nodeg(h2, srcl3, srch3, tgt3)
        acc2 = acc.reshape(NC, ACC_ROWS // 2, HIDDEN)
        h = _gru_call(acc2, dega, h, wme, wmo, bm2, W_ih, bih2, W_hh, bhh2)
    return h


# speedup vs baseline: 1.3988x; 1.2133x over previous
import functools

import jax
import jax.numpy as jnp
from jax import lax
from jax.experimental import pallas as pl
from jax.experimental.pallas import tpu as pltpu
from jax.experimental.pallas import tpu_sc as plsc

N_NODES = 10000
HIDDEN = 128
ANNOT = 16
TIMESTEPS = 2
N_EDGES = 320000

NC = 2
NS = 16
HALF = HIDDEN // NC
CHUNK = 128
NCHUNK = 160
RING = 4
EPT = CHUNK * NCHUNK
E_PAD = EPT * NS
ACC_ROWS = 10240
DEGW = 16
ROWS_PER_TILE = ACC_ROWS // NS


def _edge_agg_body(with_deg, *refs):
    if with_deg:
        (h2_hbm, srcl_hbm, srch_hbm, tgt_hbm, acc_out, deg_out,
         src_v, tgt_v, rows_v, ones_v, zdeg_v,
         acc_sh, deg_sh, *sems) = refs
    else:
        (h2_hbm, srcl_hbm, srch_hbm, tgt_hbm, acc_out,
         src_v, tgt_v, rows_v,
         acc_sh, *sems) = refs
    gsem = sems[:RING]
    ssem = sems[RING:]

    c = lax.axis_index("c")
    s = lax.axis_index("s")

    for ci, ref in enumerate((srcl_hbm, srch_hbm)):
        @pl.when(c == ci)
        def _(ref=ref):
            pltpu.sync_copy(ref.at[s], src_v)
    pltpu.sync_copy(tgt_hbm.at[s], tgt_v)

    z16 = jnp.zeros((16,), jnp.float32)
    one16 = jnp.ones((16,), jnp.float32)

    def fill_row(i, carry):
        for v in range(HALF // 16):
            rows_v[0, i, pl.ds(v * 16, 16)] = z16
        if with_deg:
            zdeg_v[i, pl.ds(0, 16)] = z16
            ones_v[i, pl.ds(0, 16)] = one16
        return carry

    lax.fori_loop(0, CHUNK, fill_row, 0)

    for b in range(ROWS_PER_TILE // CHUNK):
        base = s * ROWS_PER_TILE + b * CHUNK
        pltpu.sync_copy(rows_v.at[0], acc_sh.at[pl.ds(base, CHUNK)])
        if with_deg:
            pltpu.sync_copy(zdeg_v, deg_sh.at[pl.ds(base, CHUNK)])
    plsc.subcore_barrier()

    def fire_gather(j, a):
        pltpu.async_copy(h2_hbm.at[src_v.at[j]], rows_v.at[a], gsem[a])

    def wait_gather(j, a):
        pltpu.make_async_copy(h2_hbm.at[src_v.at[j]],
                              rows_v.at[a], gsem[a]).wait()

    def fire_scatter(j, a):
        pltpu.async_copy(rows_v.at[a], acc_sh.at[tgt_v.at[j]],
                         ssem[a], add=True)

        if with_deg:
            @pl.when(c == (a % NC))
            def _():
                pltpu.async_copy(ones_v, deg_sh.at[tgt_v.at[j]],
                                 ssem[a], add=True)

    def wait_scatter(j, a):
        pltpu.make_async_copy(rows_v.at[a], acc_sh.at[tgt_v.at[j]],
                              ssem[a]).wait()

        if with_deg:
            @pl.when(c == (a % NC))
            def _():
                pltpu.make_async_copy(ones_v, deg_sh.at[tgt_v.at[j]],
                                      ssem[a]).wait()

    for a in range(RING):
        fire_gather(a, a)

    def it_body(i, carry):
        j0 = RING * i
        for a in range(RING):
            j = j0 + a
            wait_gather(j, a)
            fire_scatter(j, a)

            @pl.when(j + RING < NCHUNK)
            def _(j=j, a=a):
                wait_scatter(j, a)
                fire_gather(j + RING, a)
        return carry

    lax.fori_loop(0, NCHUNK // RING, it_body, 0)
    for a in range(RING):
        wait_scatter(NCHUNK - RING + a, a)
    plsc.subcore_barrier()

    sl = pl.ds(s * ROWS_PER_TILE, ROWS_PER_TILE)
    pltpu.sync_copy(acc_sh.at[sl], acc_out.at[c, sl])

    if with_deg:
        pltpu.sync_copy(deg_sh.at[sl], deg_out.at[c, sl])


def _make_edge_agg(with_deg):
    out_type = [jax.ShapeDtypeStruct((NC, ACC_ROWS, HALF), jnp.float32)]
    scratch = [
        pltpu.VMEM((NCHUNK, CHUNK), jnp.int32),
        pltpu.VMEM((NCHUNK, CHUNK), jnp.int32),
        pltpu.VMEM((RING, CHUNK, HALF), jnp.float32),
    ]
    if with_deg:
        out_type.append(jax.ShapeDtypeStruct((NC, ACC_ROWS, DEGW), jnp.float32))
        scratch += [
            pltpu.VMEM((CHUNK, DEGW), jnp.float32),
            pltpu.VMEM((CHUNK, DEGW), jnp.float32),
        ]
    scratch.append(pltpu.VMEM_SHARED((ACC_ROWS, HALF), jnp.float32))
    if with_deg:
        scratch.append(pltpu.VMEM_SHARED((ACC_ROWS, DEGW), jnp.float32))
    scratch += [pltpu.SemaphoreType.DMA] * (2 * RING)
    return pl.kernel(
        functools.partial(_edge_agg_body, with_deg),
        out_type=tuple(out_type) if with_deg else out_type[0],
        mesh=plsc.VectorSubcoreMesh(core_axis_name="c", subcore_axis_name="s"),
        compiler_params=pltpu.CompilerParams(use_tc_tiling_on_sc=False),
        scratch_types=scratch,
    )


_edge_agg_deg = _make_edge_agg(True)
_edge_agg_nodeg = _make_edge_agg(False)


BLK = 2000


def _h0_body(x_ref, a_ref, wx_ref, wa_ref, b_ref, o_ref):
    o_ref[...] = (jnp.dot(x_ref[...], wx_ref[...], preferred_element_type=jnp.float32)
                  + jnp.dot(a_ref[...], wa_ref[...], preferred_element_type=jnp.float32)
                  + b_ref[...])


def _h0_call(x, annot, wx, wa, b):
    return pl.pallas_call(
        _h0_body,
        grid=(N_NODES // BLK,),
        in_specs=[
            pl.BlockSpec((BLK, HIDDEN), lambda i: (i, 0)),
            pl.BlockSpec((BLK, ANNOT), lambda i: (i, 0)),
            pl.BlockSpec((HIDDEN, HIDDEN), lambda i: (0, 0)),
            pl.BlockSpec((ANNOT, HIDDEN), lambda i: (0, 0)),
            pl.BlockSpec((1, HIDDEN), lambda i: (0, 0)),
        ],
        out_specs=pl.BlockSpec((BLK, HIDDEN), lambda i: (i, 0)),
        out_shape=jax.ShapeDtypeStruct((N_NODES, HIDDEN), jnp.float32),
    )(x, annot, wx, wa, b)


def _gru_body(acc_ref, dega_ref, h_ref, wme_ref, wmo_ref, bmsg_ref,
              wih_ref, bih_ref, whh_ref, bhh_ref, o_ref):
    deg = dega_ref[0, :, :1] + dega_ref[1, :, :1]
    q_even =(jnp.dot(acc_ref[0], wme_ref[0], preferred_element_type=jnp.float32)
              + jnp.dot(acc_ref[1], wme_ref[1], preferred_element_type=jnp.float32))
    q_odd = (jnp.dot(acc_ref[0], wmo_ref[0], preferred_element_type=jnp.float32)
             + jnp.dot(acc_ref[1], wmo_ref[1], preferred_element_type=jnp.float32))
    inc = (jnp.stack([q_even, q_odd], axis=1).reshape(BLK, HIDDEN)
           + deg * bmsg_ref[...])
    h = h_ref[...]
    gi = jnp.dot(inc, wih_ref[...], preferred_element_type=jnp.float32) + bih_ref[...]
    gh = jnp.dot(h, whh_ref[...], preferred_element_type=jnp.float32) + bhh_ref[...]
    r = jax.nn.sigmoid(gi[:, :HIDDEN] + gh[:, :HIDDEN])
    z = jax.nn.sigmoid(gi[:, HIDDEN:2 * HIDDEN] + gh[:, HIDDEN:2 * HIDDEN])
    n = jnp.tanh(gi[:, 2 * HIDDEN:] + r * gh[:, 2 * HIDDEN:])
    o_ref[...] = (1.0 - z) * n + z * h


def _gru_call(acc, dega, h, wme, wmo, bmsg, wih, bih, whh, bhh):
    return pl.pallas_call(
        _gru_body,
        grid=(N_NODES // BLK,),
        in_specs=[
            pl.BlockSpec((NC, BLK // 2, HIDDEN), lambda i: (0, i, 0)),
            pl.BlockSpec((NC, BLK, DEGW), lambda i: (0, i, 0)),
            pl.BlockSpec((BLK, HIDDEN), lambda i: (i, 0)),
            pl.BlockSpec((NC, HIDDEN, HIDDEN), lambda i: (0, 0, 0)),
            pl.BlockSpec((NC, HIDDEN, HIDDEN), lambda i: (0, 0, 0)),
            pl.BlockSpec((1, HIDDEN), lambda i: (0, 0)),
            pl.BlockSpec((HIDDEN, 3 * HIDDEN), lambda i: (0, 0)),
            pl.BlockSpec((1, 3 * HIDDEN), lambda i: (0, 0)),
            pl.BlockSpec((HIDDEN, 3 * HIDDEN), lambda i: (0, 0)),
            pl.BlockSpec((1, 3 * HIDDEN), lambda i: (0, 0)),
        ],
        out_specs=pl.BlockSpec((BLK, HIDDEN), lambda i: (i, 0)),
        out_shape=jax.ShapeDtypeStruct((N_NODES, HIDDEN), jnp.float32),
    )(acc, dega, h, wme, wmo, bmsg, wih, bih, whh, bhh)


def kernel(initial_node_representation, annotations, adjacency_lists,
           W_hidden, b_hidden, W_msg, b_msg, W_ih, b_ih, W_hh, b_hh):
    src = adjacency_lists[:, 0].astype(jnp.int32)
    tgt = adjacency_lists[:, 1].astype(jnp.int32)

    npad = E_PAD - N_EDGES
    ar = jnp.arange(npad, dtype=jnp.int32)
    pad_src = (ar * 37) % N_NODES
    pad_tgt = N_NODES + ar % (ACC_ROWS - N_NODES)
    src2 = 2 * jnp.concatenate([src, pad_src])
    srcl3 = src2.reshape(NS, NCHUNK, CHUNK)
    srch3 = (src2 + 1).reshape(NS, NCHUNK, CHUNK)
    tgt3 = jnp.concatenate([tgt, pad_tgt]).reshape(NS, NCHUNK, CHUNK)

    wx = W_hidden[:HIDDEN]
    wa = W_hidden[HIDDEN:]
    z64 = jnp.zeros((HALF, HIDDEN), jnp.float32)
    wme = jnp.stack([jnp.concatenate([W_msg[:HALF], z64]),
                     jnp.concatenate([W_msg[HALF:], z64])])
    wmo = jnp.stack([jnp.concatenate([z64, W_msg[:HALF]]),
                     jnp.concatenate([z64, W_msg[HALF:]])])
    bh2 = b_hidden.reshape(1, HIDDEN)
    bm2 = b_msg.reshape(1, HIDDEN)
    bih2 = b_ih.reshape(1, 3 * HIDDEN)
    bhh2 = b_hh.reshape(1, 3 * HIDDEN)

    h = _h0_call(initial_node_representation, annotations, wx, wa, bh2)
    dega = None
    for t in range(TIMESTEPS):
        h2 = h.reshape(2 * N_NODES, HALF)
        if t == 0:
            acc, dega = _edge_agg_deg(h2, srcl3, srch3, tgt3)
        else:
            acc = _edge_agg_nodeg(h2, srcl3, srch3, tgt3)
        acc2 = acc.reshape(NC, ACC_ROWS // 2, HIDDEN)
        h = _gru_call(acc2, dega, h, wme, wmo, bm2, W_ih, bih2, W_hh, bhh2)
    return h
